# baseline (speedup 1.0000x reference)
import jax
import jax.numpy as jnp
from jax.experimental import pallas as pl
from jax.experimental.pallas import tpu as pltpu

_B, _C, _H, _W = 8, 17, 512, 512
_ROWS = 512
_SUB = 8
_PAIR = 8
_GRID = (_B, _H // _ROWS)
_NBLK = _GRID[0] * _GRID[1]
_NPIX = float(_B * _H * _W)


def _dice_body(smooth_ref, x_ref, t_ref, out_ref, accD_ref, accI_ref):
    b = pl.program_id(0)
    i = pl.program_id(1)
    pid = b * _GRID[1] + i

    @pl.when(pid == 0)
    def _init():
        accD_ref[...] = jnp.zeros_like(accD_ref)
        accI_ref[...] = jnp.zeros_like(accI_ref)

    cls = jax.lax.broadcasted_iota(jnp.int32, (_C, _SUB, _W), 0)
    for g in range(_ROWS // (_SUB * _PAIR)):
        dps = []
        ips = []
        for p in range(_PAIR):
            k = g * _PAIR + p
            xk = x_ref[0, :, pl.ds(k * _SUB, _SUB), :]
            tk = t_ref[0, pl.ds(k * _SUB, _SUB), :]
            Tk = jnp.tanh(0.5 * xk)
            mf2 = jnp.where(cls == tk[None], 2.0, 0.0)
            dps.append(Tk + mf2)
            ips.append(jnp.sum(Tk * mf2, axis=0))
        accD_ref[...] += sum(dps)
        accI_ref[...] += sum(ips)

    @pl.when(pid == _NBLK - 1)
    def _finish():
        smooth = smooth_ref[0, 0]
        denom = 0.5 * jnp.sum(accD_ref[...], axis=(1, 2)) + (0.5 * _NPIX)
        inter = 0.25 * jnp.sum(accI_ref[...]) + (0.5 * _NPIX)
        out_ref[0, 0] = -(2.0 * inter + jnp.sum(smooth / (denom + smooth)))


def kernel(input, target, smooth):
    smooth2d = jnp.reshape(smooth, (1, 1)).astype(jnp.float32)
    out = pl.pallas_call(
        _dice_body,
        grid=_GRID,
        in_specs=[
            pl.BlockSpec(memory_space=pltpu.SMEM),
            pl.BlockSpec((1, _C, _ROWS, _W), lambda b, i: (b, 0, i, 0)),
            pl.BlockSpec((1, _ROWS, _W), lambda b, i: (b, i, 0)),
        ],
        out_specs=pl.BlockSpec(memory_space=pltpu.SMEM),
        out_shape=jax.ShapeDtypeStruct((1, 1), jnp.float32),
        scratch_shapes=[
            pltpu.VMEM((_C, _SUB, _W), jnp.float32),
            pltpu.VMEM((_SUB, _W), jnp.float32),
        ],
    )(smooth2d, input, target)
    return out[0, 0]


# speedup vs baseline: 1.0003x; 1.0003x over previous
import jax
import jax.numpy as jnp
from jax.experimental import pallas as pl
from jax.experimental.pallas import tpu as pltpu

_B, _C, _H, _W = 8, 17, 512, 512
_ROWS = 512
_SUB = 8
_PAIR = 4
_GRID = (_B, _H // _ROWS)
_NBLK = _GRID[0] * _GRID[1]
_NPIX = float(_B * _H * _W)


def _dice_body(smooth_ref, x_ref, t_ref, out_ref, accD_ref, accI_ref):
    b = pl.program_id(0)
    i = pl.program_id(1)
    pid = b * _GRID[1] + i

    @pl.when(pid == 0)
    def _init():
        accD_ref[...] = jnp.zeros_like(accD_ref)
        accI_ref[...] = jnp.zeros_like(accI_ref)

    cls = jax.lax.broadcasted_iota(jnp.int32, (_C, _SUB, _W), 0)
    for g in range(_ROWS // (_SUB * _PAIR)):
        dps = []
        ips = []
        for p in range(_PAIR):
            k = g * _PAIR + p
            xk = x_ref[0, :, pl.ds(k * _SUB, _SUB), :]
            tk = t_ref[0, pl.ds(k * _SUB, _SUB), :]
            Tk = jnp.tanh(0.5 * xk)
            mf2 = jnp.where(cls == tk[None], 2.0, 0.0)
            dps.append(Tk + mf2)
            ips.append(jnp.sum(Tk * mf2, axis=0))
        accD_ref[...] += sum(dps)
        accI_ref[...] += sum(ips)

    @pl.when(pid == _NBLK - 1)
    def _finish():
        smooth = smooth_ref[0, 0]
        denom = 0.5 * jnp.sum(accD_ref[...], axis=(1, 2)) + (0.5 * _NPIX)
        inter = 0.25 * jnp.sum(accI_ref[...]) + (0.5 * _NPIX)
        out_ref[0, 0] = -(2.0 * inter + jnp.sum(smooth / (denom + smooth)))


def kernel(input, target, smooth):
    smooth2d = jnp.reshape(smooth, (1, 1)).astype(jnp.float32)
    out = pl.pallas_call(
        _dice_body,
        grid=_GRID,
        in_specs=[
            pl.BlockSpec(memory_space=pltpu.SMEM),
            pl.BlockSpec((1, _C, _ROWS, _W), lambda b, i: (b, 0, i, 0)),
            pl.BlockSpec((1, _ROWS, _W), lambda b, i: (b, i, 0)),
        ],
        out_specs=pl.BlockSpec(memory_space=pltpu.SMEM),
        out_shape=jax.ShapeDtypeStruct((1, 1), jnp.float32),
        scratch_shapes=[
            pltpu.VMEM((_C, _SUB, _W), jnp.float32),
            pltpu.VMEM((_SUB, _W), jnp.float32),
        ],
    )(smooth2d, input, target)
    return out[0, 0]
